# Initial kernel scaffold; baseline (speedup 1.0000x reference)
import jax
import jax.numpy as jnp
from jax.experimental import pallas as pl


def kernel(database, query, input_features, filter_weights, nn_index, nn_count, nn_dist):
    raise NotImplementedError("write your pallas kernel here")


# speedup vs baseline: 10.2476x; 10.2476x over previous
import functools
import math

import jax
import jax.numpy as jnp
from jax import lax
from jax.experimental import pallas as pl
from jax.experimental.pallas import tpu as pltpu
from jax.experimental.pallas import tpu_sc as plsc

N_AZ, N_EL, N_RAD = 4, 2, 2
RADIUS = 0.05
AZ_SCALE = N_AZ / (2.0 * math.pi)
EL_SCALE = N_EL / math.pi
NBINS = N_AZ * N_EL * N_RAD

ROW = 32
CHUNK = 128
MB = 256


def _sc_gather(table, flat_idx, n_rows):
    info = plsc.get_sparse_core_info()
    ncores = info.num_cores
    nw = ncores * info.num_subcores
    per_w = n_rows // nw
    n_chunks = per_w // CHUNK
    mesh = plsc.VectorSubcoreMesh(core_axis_name="c", subcore_axis_name="s")

    @functools.partial(
        pl.kernel,
        mesh=mesh,
        compiler_params=pltpu.CompilerParams(use_tc_tiling_on_sc=False),
        out_type=jax.ShapeDtypeStruct((n_rows // CHUNK, CHUNK, ROW),
                                      jnp.float32),
        scratch_types=[
            pltpu.VMEM((n_chunks, CHUNK), jnp.int32),
            pltpu.VMEM((CHUNK, ROW), jnp.float32),
            pltpu.SemaphoreType.DMA,
        ],
    )
    def gather_kernel(table_hbm, idx_hbm, out_hbm, idx_v, rows_v, sem):
        wid = lax.axis_index("s") * ncores + lax.axis_index("c")
        pltpu.sync_copy(idx_hbm.at[wid], idx_v)

        def body(j, carry):
            pltpu.async_copy(table_hbm.at[idx_v.at[j]], rows_v, sem).wait()
            pltpu.sync_copy(rows_v, out_hbm.at[wid * n_chunks + j])
            return carry

        lax.fori_loop(0, n_chunks, body, 0)

    return gather_kernel(table, flat_idx.reshape(nw, n_chunks, CHUNK))


def _tc_combine(g_ref, q_ref, dist_ref, w_ref, out_ref):
    mb, kr = g_ref.shape[1], g_ref.shape[2]
    k = kr // ROW
    g = g_ref[0].reshape(mb, k, ROW)
    q = q_ref[0]
    dist = dist_ref[0]

    x = g[:, :, 0] - q[:, 0][:, None]
    y = g[:, :, 1] - q[:, 1][:, None]
    z = g[:, :, 2] - q[:, 2][:, None]
    feat = g[:, :, 16:]

    azimuth = jnp.arctan2(y, x) + math.pi
    t = jnp.clip(z / (dist + 1e-8), -1.0, 1.0)
    elevation = jnp.arctan2(jnp.sqrt(jnp.maximum(1.0 - t * t, 0.0)), t)

    az_bin = azimuth * AZ_SCALE
    el_bin = elevation * EL_SCALE
    r_bin = jnp.clip(dist / RADIUS, 0.0, N_RAD - 1e-6)
    a_f = jnp.floor(az_bin)
    e_f = jnp.floor(el_bin)
    r_f = jnp.floor(r_bin)
    a_frac = az_bin - a_f
    e_frac = el_bin - e_f
    r_frac = r_bin - r_f
    a0 = a_f.astype(jnp.int32)
    e0 = e_f.astype(jnp.int32)
    r0 = r_f.astype(jnp.int32)

    e_c = jnp.clip(e0, 0, N_EL - 1)
    e_c1 = jnp.clip(e0 + 1, 0, N_EL - 1)
    r_c = jnp.clip(r0, 0, N_RAD - 1)
    r_c1 = jnp.clip(r0 + 1, 0, N_RAD - 1)
    base_a = (a0 % N_AZ) * N_EL
    base_a1 = ((a0 + 1) % N_AZ) * N_EL

    a_i = 1.0 - a_frac
    e_i = 1.0 - e_frac
    r_i = 1.0 - r_frac
    corners = (
        (a_i * e_i * r_i, (base_a + e_c) * N_RAD + r_c),
        (a_frac * e_i * r_i, (base_a1 + e_c) * N_RAD + r_c),
        (a_i * e_frac * r_i, (base_a + e_c1) * N_RAD + r_c),
        (a_frac * e_frac * r_i, (base_a1 + e_c1) * N_RAD + r_c),
        (a_i * e_i * r_frac, (base_a + e_c) * N_RAD + r_c1),
        (a_frac * e_i * r_frac, (base_a1 + e_c) * N_RAD + r_c1),
        (a_i * e_frac * r_frac, (base_a + e_c1) * N_RAD + r_c1),
        (a_frac * e_frac * r_frac, (base_a1 + e_c1) * N_RAD + r_c1),
    )
    bins = lax.broadcasted_iota(jnp.int32, (mb, k, NBINS), 2)
    coeff16 = jnp.zeros((mb, k, NBINS), jnp.float32)
    for cf, bi in corners:
        coeff16 = coeff16 + jnp.where(bins == bi[:, :, None],
                                      cf[:, :, None], 0.0)

    w_eff = lax.dot_general(
        coeff16.reshape(mb * k, NBINS), w_ref[...],
        (((1,), (0,)), ((), ())),
        preferred_element_type=jnp.float32,
    ).reshape(mb, k, -1)
    out_ref[0] = jnp.sum(feat * w_eff, axis=1)


def kernel(database, query, input_features, filter_weights, nn_index,
           nn_count, nn_dist):
    B, M, K = nn_index.shape
    N = database.shape[1]
    C = input_features.shape[-1]

    table = jnp.zeros((B * N, ROW), jnp.float32)
    table = table.at[:, 0:3].set(database.reshape(B * N, 3))
    table = table.at[:, 16:16 + C].set(input_features.reshape(B * N, C))
    flat_idx = (nn_index
                + (jnp.arange(B, dtype=jnp.int32) * N)[:, None, None]
                ).reshape(-1)

    gathered = _sc_gather(table, flat_idx, B * M * K)
    gathered = gathered.reshape(B, M, K * ROW)

    out = pl.pallas_call(
        _tc_combine,
        grid=(B, M // MB),
        in_specs=[
            pl.BlockSpec((1, MB, K * ROW), lambda b, i: (b, i, 0)),
            pl.BlockSpec((1, MB, 3), lambda b, i: (b, i, 0)),
            pl.BlockSpec((1, MB, K), lambda b, i: (b, i, 0)),
            pl.BlockSpec((NBINS, C), lambda b, i: (0, 0)),
        ],
        out_specs=pl.BlockSpec((1, MB, C), lambda b, i: (b, i, 0)),
        out_shape=jax.ShapeDtypeStruct((B, M, C), jnp.float32),
    )(gathered, query, nn_dist, filter_weights.reshape(NBINS, C))
    return out
